# Initial kernel scaffold; baseline (speedup 1.0000x reference)
import jax
import jax.numpy as jnp
from jax.experimental import pallas as pl


def kernel(edge_index, edge_type, init_embed, init_rel, w_loop1, w_in1, w_out1, w_rel1, w_loop2, w_in2, w_out2, w_rel2, loop_rel1, loop_rel2, bias1, bias2, bn1_gamma, bn1_beta, bn2_gamma, bn2_beta):
    raise NotImplementedError("write your pallas kernel here")


# speedup vs baseline: 5.7322x; 5.7322x over previous
import functools
import math

import jax
import jax.numpy as jnp
from jax import lax
from jax.experimental import pallas as pl
from jax.experimental.pallas import tpu as pltpu
from jax.experimental.pallas import tpu_sc as plsc

NUM_ENT = 10000
NUM_RELROWS = 400
DIM = 128
NUM_EDGES = 320000
NE_HALF = NUM_EDGES // 2

NC = 2
NS = 16
K = 128
EPT = NE_HALF // NS
CH = 80
EPT_PAD = CH * K
PAD_ROWS = 112
ENT_PAD = NUM_ENT + PAD_ROWS
STRIPE = ENT_PAD // NS


@functools.cache
def _sc_mesh():
    return plsc.VectorSubcoreMesh(core_axis_name="c", subcore_axis_name="s",
                                  num_cores=NC, num_subcores=NS)


def _memset_rows(ref, nrows, ncols, value):
    def body(i, _):
        for cc in range(ncols // 16):
            ref[i, pl.ds(cc * 16, 16)] = jnp.full((16,), value, jnp.float32)
        return 0
    lax.fori_loop(0, nrows, body, 0)


def _zero_stripe(s, zbuf, shared):
    for m in range(NS):
        @pl.when(s == m)
        def _():
            row = m * STRIPE
            for n in (K, K, K, K, STRIPE - 4 * K):
                pltpu.sync_copy(zbuf.at[pl.ds(0, n)],
                                shared.at[pl.ds(row, n)])
                row += n


def _stripe_out(s, c, shared, out_hbm):
    for m in range(NS):
        @pl.when(s == m)
        def _():
            row = m * STRIPE
            for n in (K, K, K, K, STRIPE - 4 * K):
                pltpu.sync_copy(shared.at[pl.ds(row, n)],
                                out_hbm.at[c, pl.ds(row, n)])
                row += n


@functools.cache
def _deg_kernel_fn():
    return functools.partial(
        pl.kernel,
        out_type=jax.ShapeDtypeStruct((NC, ENT_PAD, DIM), jnp.float32),
        mesh=_sc_mesh(),
        scratch_types=[
            pltpu.VMEM((K,), jnp.int32),
            pltpu.VMEM((K,), jnp.int32),
            pltpu.VMEM((K, DIM), jnp.float32),
            pltpu.SemaphoreType.DMA,
            pltpu.VMEM_SHARED((ENT_PAD, DIM), jnp.float32),
        ],
    )(_deg_body)


def _deg_kernel(agg_all):
    return _deg_kernel_fn()(agg_all)


def _deg_body(agg_hbm, out_hbm, agg_v0, agg_v1, ones_v, semi, shared_deg):
    c = lax.axis_index("c")
    s = lax.axis_index("s")
    _memset_rows(ones_v, K, DIM, 0.0)
    _zero_stripe(s, ones_v, shared_deg)
    _memset_rows(ones_v, K, DIM, 1.0)
    pltpu.sync_copy(agg_hbm.at[c, s, 0], agg_v0)
    plsc.subcore_barrier()

    def pair(j2, _):
        j0 = j2 * 2
        p = pltpu.async_copy(agg_hbm.at[c, s, j0 + 1], agg_v1, semi)
        pltpu.sync_copy(ones_v, shared_deg.at[agg_v0], add=True)
        p.wait()
        jn = jnp.minimum(j0 + 2, CH - 1)
        q = pltpu.async_copy(agg_hbm.at[c, s, jn], agg_v0, semi)
        pltpu.sync_copy(ones_v, shared_deg.at[agg_v1], add=True)
        q.wait()
        return 0
    lax.fori_loop(0, CH // 2, pair, 0)

    plsc.subcore_barrier()
    _stripe_out(s, c, shared_deg, out_hbm)


@functools.cache
def _agg_kernel_fn():
    return functools.partial(
        pl.kernel,
        out_type=jax.ShapeDtypeStruct((NC, ENT_PAD, DIM), jnp.float32),
        mesh=_sc_mesh(),
        scratch_types=[
            pltpu.VMEM((K,), jnp.int32),
            pltpu.VMEM((K,), jnp.int32),
            pltpu.VMEM((K,), jnp.int32),
            pltpu.VMEM((K,), jnp.int32),
            pltpu.VMEM((K,), jnp.int32),
            pltpu.VMEM((K,), jnp.int32),
            pltpu.VMEM((K, DIM), jnp.float32),
            pltpu.VMEM((K, DIM), jnp.float32),
            pltpu.SemaphoreType.DMA,
            pltpu.SemaphoreType.DMA,
            pltpu.SemaphoreType.DMA,
            pltpu.VMEM_SHARED((ENT_PAD, DIM), jnp.float32),
        ],
    )(_agg_body)


def _agg_kernel(agg_all, src_all, typ_all, xs, rel):
    return _agg_kernel_fn()(agg_all, src_all, typ_all, xs, rel)


def _agg_chunk(c, s, cur, other, jnext, agg_hbm, src_hbm, typ_hbm,
               xs_hbm, rel_hbm, xj_v, rl_v, sem1, sem2, semi, shared_s):
    agg_c, src_c, typ_c = cur
    agg_o, src_o, typ_o = other
    p1 = pltpu.async_copy(agg_hbm.at[c, s, jnext], agg_o, semi)
    p2 = pltpu.async_copy(src_hbm.at[c, s, jnext], src_o, semi)
    p3 = pltpu.async_copy(typ_hbm.at[c, s, jnext], typ_o, semi)
    g1 = pltpu.async_copy(xs_hbm.at[src_c], xj_v, sem1)
    g2 = pltpu.async_copy(rel_hbm.at[typ_c], rl_v, sem2)
    g1.wait()
    g2.wait()

    def rows(e, _):
        base = e * 4
        for u in range(4):
            for cc in range(DIM // 16):
                sl = pl.ds(cc * 16, 16)
                xj_v[base + u, sl] = xj_v[base + u, sl] * rl_v[base + u, sl]
        return 0
    lax.fori_loop(0, K // 4, rows, 0)
    pltpu.sync_copy(xj_v, shared_s.at[agg_c], add=True)
    p1.wait()
    p2.wait()
    p3.wait()


def _agg_body(agg_hbm, src_hbm, typ_hbm, xs_hbm, rel_hbm, out_hbm,
              agg_v0, agg_v1, src_v0, src_v1, typ_v0, typ_v1,
              xj_v, rl_v, sem1, sem2, semi, shared_s):
    c = lax.axis_index("c")
    s = lax.axis_index("s")
    _memset_rows(xj_v, K, DIM, 0.0)
    _zero_stripe(s, xj_v, shared_s)
    pltpu.sync_copy(agg_hbm.at[c, s, 0], agg_v0)
    pltpu.sync_copy(src_hbm.at[c, s, 0], src_v0)
    pltpu.sync_copy(typ_hbm.at[c, s, 0], typ_v0)
    plsc.subcore_barrier()

    slot0 = (agg_v0, src_v0, typ_v0)
    slot1 = (agg_v1, src_v1, typ_v1)

    def pair(j2, _):
        j0 = j2 * 2
        jn = jnp.minimum(j0 + 2, CH - 1)
        args = (agg_hbm, src_hbm, typ_hbm, xs_hbm, rel_hbm,
                xj_v, rl_v, sem1, sem2, semi, shared_s)
        _agg_chunk(c, s, slot0, slot1, j0 + 1, *args)
        _agg_chunk(c, s, slot1, slot0, jn, *args)
        return 0
    lax.fori_loop(0, CH // 2, pair, 0)

    plsc.subcore_barrier()
    _stripe_out(s, c, shared_s, out_hbm)


_RB = 1000


def _dinv_col(deg_blk):
    d = deg_blk[:, 0:1]
    return jnp.where(d > 0, lax.rsqrt(jnp.maximum(d, 1e-12)), 0.0)


def _prep_body(x_ref, dgi_ref, dgo_ref, out_ref):
    di = _dinv_col(dgi_ref[...])
    do = _dinv_col(dgo_ref[...])
    x = x_ref[...]
    out_ref[0, :, :] = x * di
    out_ref[1, :, :] = x * do


def _prep_scaled(x, deg_in, deg_out):
    return pl.pallas_call(
        _prep_body,
        grid=(NUM_ENT // _RB,),
        in_specs=[
            pl.BlockSpec((_RB, DIM), lambda i: (i, 0)),
            pl.BlockSpec((_RB, DIM), lambda i: (i, 0)),
            pl.BlockSpec((_RB, DIM), lambda i: (i, 0)),
        ],
        out_specs=pl.BlockSpec((2, _RB, DIM), lambda i: (0, i, 0)),
        out_shape=jax.ShapeDtypeStruct((2, NUM_ENT, DIM), jnp.float32),
    )(x, deg_in, deg_out)


_BN_RS = 1.0 / math.sqrt(1.0 + 1e-5)


def _layer_body(emit_scaled, s_in_ref, s_out_ref, x_ref, dgi_ref, dgo_ref,
                w_in_ref, w_out_ref, w_loop_ref, lrel_ref, bias_ref,
                bng_ref, bnb_ref, *out_refs):
    di = _dinv_col(dgi_ref[...])
    do = _dinv_col(dgo_ref[...])
    x = x_ref[...]
    a = jnp.dot(s_in_ref[...] * di, w_in_ref[...],
                preferred_element_type=jnp.float32)
    a = a + jnp.dot(s_out_ref[...] * do, w_out_ref[...],
                    preferred_element_type=jnp.float32)
    a = a + jnp.dot(x * lrel_ref[...], w_loop_ref[...],
                    preferred_element_type=jnp.float32)
    h = a * (1.0 / 3.0) + bias_ref[...]
    h = (h * _BN_RS) * bng_ref[...] + bnb_ref[...]
    x2 = jnp.tanh(h)
    out_refs[0][...] = x2
    if emit_scaled:
        out_refs[1][0, :, :] = x2 * di
        out_refs[1][1, :, :] = x2 * do


def _layer(emit_scaled, s_in, s_out, x, deg_in, deg_out,
           w_in, w_out, w_loop, loop_rel, bias, bn_g, bn_b):
    full = lambda r, c: pl.BlockSpec((r, c), lambda i: (0, 0))
    out_shape = [jax.ShapeDtypeStruct((NUM_ENT, DIM), jnp.float32)]
    out_specs = [pl.BlockSpec((_RB, DIM), lambda i: (i, 0))]
    if emit_scaled:
        out_shape.append(jax.ShapeDtypeStruct((2, NUM_ENT, DIM), jnp.float32))
        out_specs.append(pl.BlockSpec((2, _RB, DIM), lambda i: (0, i, 0)))
    return pl.pallas_call(
        functools.partial(_layer_body, emit_scaled),
        grid=(NUM_ENT // _RB,),
        in_specs=[
            pl.BlockSpec((_RB, DIM), lambda i: (i, 0)),
            pl.BlockSpec((_RB, DIM), lambda i: (i, 0)),
            pl.BlockSpec((_RB, DIM), lambda i: (i, 0)),
            pl.BlockSpec((_RB, DIM), lambda i: (i, 0)),
            pl.BlockSpec((_RB, DIM), lambda i: (i, 0)),
            full(DIM, DIM), full(DIM, DIM), full(DIM, DIM),
            full(1, DIM), full(1, DIM), full(1, DIM), full(1, DIM),
        ],
        out_specs=out_specs,
        out_shape=out_shape,
    )(s_in, s_out, x, deg_in, deg_out, w_in, w_out, w_loop,
      loop_rel, bias, bn_g, bn_b)


def _relmm_body(r_ref, w_ref, o_ref):
    o_ref[...] = jnp.dot(r_ref[...], w_ref[...],
                         preferred_element_type=jnp.float32)


def _relmm(r, w):
    return pl.pallas_call(
        _relmm_body,
        out_shape=jax.ShapeDtypeStruct((NUM_RELROWS, DIM), jnp.float32),
    )(r, w)


def _prep_half(agg, src, typ, src_offset):
    def pad(a, fill):
        a = a.reshape(NS, EPT)
        a = jnp.pad(a, ((0, 0), (0, EPT_PAD - EPT)), constant_values=fill)
        return a.reshape(NS, CH, K)
    return (pad(agg, NUM_ENT), pad(src + src_offset, src_offset), pad(typ, 0))


def kernel(edge_index, edge_type, init_embed, init_rel,
           w_loop1, w_in1, w_out1, w_rel1,
           w_loop2, w_in2, w_out2, w_rel2,
           loop_rel1, loop_rel2, bias1, bias2,
           bn1_gamma, bn1_beta, bn2_gamma, bn2_beta):
    ei = edge_index.astype(jnp.int32)
    et = edge_type.astype(jnp.int32)
    agg_i, src_i, typ_i = _prep_half(ei[0, :NE_HALF], ei[1, :NE_HALF],
                                     et[:NE_HALF], 0)
    agg_o, src_o, typ_o = _prep_half(ei[0, NE_HALF:], ei[1, NE_HALF:],
                                     et[NE_HALF:], NUM_ENT)
    agg_all = jnp.stack([agg_i, agg_o])
    src_all = jnp.stack([src_i, src_o])
    typ_all = jnp.stack([typ_i, typ_o])

    deg2 = _deg_kernel(agg_all)
    deg_in = deg2[0, :NUM_ENT]
    deg_out = deg2[1, :NUM_ENT]

    b1 = bias1.reshape(1, DIM)
    b2 = bias2.reshape(1, DIM)
    g1 = bn1_gamma.reshape(1, DIM)
    be1 = bn1_beta.reshape(1, DIM)
    g2 = bn2_gamma.reshape(1, DIM)
    be2 = bn2_beta.reshape(1, DIM)

    xs1 = _prep_scaled(init_embed, deg_in, deg_out).reshape(2 * NUM_ENT, DIM)
    s1 = _agg_kernel(agg_all, src_all, typ_all, xs1, init_rel)
    x2, xs2 = _layer(True, s1[0, :NUM_ENT], s1[1, :NUM_ENT], init_embed,
                     deg_in, deg_out, w_in1, w_out1, w_loop1,
                     loop_rel1, b1, g1, be1)
    r1 = _relmm(init_rel, w_rel1)

    s2 = _agg_kernel(agg_all, src_all, typ_all,
                     xs2.reshape(2 * NUM_ENT, DIM), r1)
    (x3,) = _layer(False, s2[0, :NUM_ENT], s2[1, :NUM_ENT], x2,
                   deg_in, deg_out, w_in2, w_out2, w_loop2,
                   loop_rel2, b2, g2, be2)
    r2 = _relmm(r1, w_rel2)
    return x3, r2


# speedup vs baseline: 5.7895x; 1.0100x over previous
import functools
import math

import jax
import jax.numpy as jnp
from jax import lax
from jax.experimental import pallas as pl
from jax.experimental.pallas import tpu as pltpu
from jax.experimental.pallas import tpu_sc as plsc

NUM_ENT = 10000
NUM_RELROWS = 400
DIM = 128
NUM_EDGES = 320000
NE_HALF = NUM_EDGES // 2

NC = 2
NS = 16
K = 80
EPT = NE_HALF // NS
CH = 128
EPT_PAD = CH * K
PAD_ROWS = 112
ENT_PAD = NUM_ENT + PAD_ROWS
STRIPE = ENT_PAD // NS


@functools.cache
def _sc_mesh():
    return plsc.VectorSubcoreMesh(core_axis_name="c", subcore_axis_name="s",
                                  num_cores=NC, num_subcores=NS)


def _memset_rows(ref, nrows, ncols, value):
    def body(i, _):
        for cc in range(ncols // 16):
            ref[i, pl.ds(cc * 16, 16)] = jnp.full((16,), value, jnp.float32)
        return 0
    lax.fori_loop(0, nrows, body, 0)


_PIECES = [K] * (STRIPE // K) + ([STRIPE % K] if STRIPE % K else [])


def _zero_stripe(s, zbuf, shared):
    for m in range(NS):
        @pl.when(s == m)
        def _():
            row = m * STRIPE
            for n in _PIECES:
                pltpu.sync_copy(zbuf.at[pl.ds(0, n)],
                                shared.at[pl.ds(row, n)])
                row += n


def _stripe_out(s, c, shared, out_hbm):
    for m in range(NS):
        @pl.when(s == m)
        def _():
            row = m * STRIPE
            for n in _PIECES:
                pltpu.sync_copy(shared.at[pl.ds(row, n)],
                                out_hbm.at[c, pl.ds(row, n)])
                row += n


@functools.cache
def _deg_kernel_fn():
    return functools.partial(
        pl.kernel,
        out_type=jax.ShapeDtypeStruct((NC, ENT_PAD, DIM), jnp.float32),
        mesh=_sc_mesh(),
        scratch_types=[
            pltpu.VMEM((K,), jnp.int32),
            pltpu.VMEM((K,), jnp.int32),
            pltpu.VMEM((K, DIM), jnp.float32),
            pltpu.SemaphoreType.DMA,
            pltpu.VMEM_SHARED((ENT_PAD, DIM), jnp.float32),
        ],
    )(_deg_body)


def _deg_kernel(agg_all):
    return _deg_kernel_fn()(agg_all)


def _deg_body(agg_hbm, out_hbm, agg_v0, agg_v1, ones_v, semi, shared_deg):
    c = lax.axis_index("c")
    s = lax.axis_index("s")
    _memset_rows(ones_v, K, DIM, 0.0)
    _zero_stripe(s, ones_v, shared_deg)
    _memset_rows(ones_v, K, DIM, 1.0)
    pltpu.sync_copy(agg_hbm.at[c, s, 0], agg_v0)
    plsc.subcore_barrier()

    def pair(j2, _):
        j0 = j2 * 2
        p = pltpu.async_copy(agg_hbm.at[c, s, j0 + 1], agg_v1, semi)
        pltpu.sync_copy(ones_v, shared_deg.at[agg_v0], add=True)
        p.wait()
        jn = jnp.minimum(j0 + 2, CH - 1)
        q = pltpu.async_copy(agg_hbm.at[c, s, jn], agg_v0, semi)
        pltpu.sync_copy(ones_v, shared_deg.at[agg_v1], add=True)
        q.wait()
        return 0
    lax.fori_loop(0, CH // 2, pair, 0)

    plsc.subcore_barrier()
    _stripe_out(s, c, shared_deg, out_hbm)


@functools.cache
def _agg_kernel_fn():
    idx_slots = []
    for _ in range(4):
        idx_slots += [pltpu.VMEM((K,), jnp.int32)] * 3
        idx_slots += [pltpu.SemaphoreType.DMA]
    return functools.partial(
        pl.kernel,
        out_type=jax.ShapeDtypeStruct((NC, ENT_PAD, DIM), jnp.float32),
        mesh=_sc_mesh(),
        scratch_types=idx_slots + [
            pltpu.VMEM((K, DIM), jnp.float32),
            pltpu.VMEM((K, DIM), jnp.float32),
            pltpu.SemaphoreType.DMA,
            pltpu.SemaphoreType.DMA,
            pltpu.VMEM((K, DIM), jnp.float32),
            pltpu.VMEM((K, DIM), jnp.float32),
            pltpu.SemaphoreType.DMA,
            pltpu.SemaphoreType.DMA,
            pltpu.VMEM_SHARED((ENT_PAD, DIM), jnp.float32),
        ],
    )(_agg_body)


def _agg_kernel(agg_all, src_all, typ_all, xs, rel):
    return _agg_kernel_fn()(agg_all, src_all, typ_all, xs, rel)


def _idx_issue(c, s, j, slot, agg_hbm, src_hbm, typ_hbm):
    pltpu.async_copy(agg_hbm.at[c, s, j], slot[0], slot[3])
    pltpu.async_copy(src_hbm.at[c, s, j], slot[1], slot[3])
    pltpu.async_copy(typ_hbm.at[c, s, j], slot[2], slot[3])


def _idx_drain(c, s, slot, agg_hbm, src_hbm, typ_hbm):
    pltpu.make_async_copy(agg_hbm.at[c, s, 0], slot[0], slot[3]).wait()
    pltpu.make_async_copy(src_hbm.at[c, s, 0], slot[1], slot[3]).wait()
    pltpu.make_async_copy(typ_hbm.at[c, s, 0], slot[2], slot[3]).wait()


def _agg_body(agg_hbm, src_hbm, typ_hbm, xs_hbm, rel_hbm, out_hbm, *scr):
    slots = [scr[4 * i:4 * i + 4] for i in range(4)]
    bufs = [scr[16:20], scr[20:24]]
    shared_s = scr[24]
    c = lax.axis_index("c")
    s = lax.axis_index("s")
    _memset_rows(bufs[0][0], K, DIM, 0.0)
    _zero_stripe(s, bufs[0][0], shared_s)
    pltpu.sync_copy(agg_hbm.at[c, s, 0], slots[0][0])
    pltpu.sync_copy(src_hbm.at[c, s, 0], slots[0][1])
    pltpu.sync_copy(typ_hbm.at[c, s, 0], slots[0][2])
    _idx_issue(c, s, 1, slots[1], agg_hbm, src_hbm, typ_hbm)
    pltpu.async_copy(xs_hbm.at[slots[0][1]], bufs[0][0], bufs[0][2])
    pltpu.async_copy(rel_hbm.at[slots[0][2]], bufs[0][1], bufs[0][3])
    plsc.subcore_barrier()

    def quad(q, _):
        for u in range(4):
            j = q * 4 + u
            cur = slots[u]
            nxt = slots[(u + 1) % 4]
            pre = slots[(u + 2) % 4]
            curbuf = bufs[u % 2]
            nxtbuf = bufs[(u + 1) % 2]
            _idx_drain(c, s, nxt, agg_hbm, src_hbm, typ_hbm)
            _idx_issue(c, s, jnp.minimum(j + 2, CH - 1), pre,
                       agg_hbm, src_hbm, typ_hbm)
            pltpu.async_copy(xs_hbm.at[nxt[1]], nxtbuf[0], nxtbuf[2])
            pltpu.async_copy(rel_hbm.at[nxt[2]], nxtbuf[1], nxtbuf[3])
            pltpu.make_async_copy(xs_hbm.at[cur[1]], curbuf[0],
                                  curbuf[2]).wait()
            pltpu.make_async_copy(rel_hbm.at[cur[2]], curbuf[1],
                                  curbuf[3]).wait()
            xj_v, rl_v = curbuf[0], curbuf[1]

            def rows(e, _):
                base = e * 4
                for uu in range(4):
                    for cc in range(DIM // 16):
                        sl = pl.ds(cc * 16, 16)
                        xj_v[base + uu, sl] = (xj_v[base + uu, sl]
                                               * rl_v[base + uu, sl])
                return 0
            lax.fori_loop(0, K // 4, rows, 0)
            pltpu.sync_copy(xj_v, shared_s.at[cur[0]], add=True)
        return 0
    lax.fori_loop(0, CH // 4, quad, 0)

    _idx_drain(c, s, slots[1], agg_hbm, src_hbm, typ_hbm)
    pltpu.make_async_copy(xs_hbm.at[slots[0][1]], bufs[0][0], bufs[0][2]).wait()
    pltpu.make_async_copy(rel_hbm.at[slots[0][2]], bufs[0][1], bufs[0][3]).wait()

    plsc.subcore_barrier()
    _stripe_out(s, c, shared_s, out_hbm)


_RB = 1000


def _dinv_col(deg_blk):
    d = deg_blk[:, 0:1]
    return jnp.where(d > 0, lax.rsqrt(jnp.maximum(d, 1e-12)), 0.0)


def _prep_body(x_ref, dgi_ref, dgo_ref, out_ref):
    di = _dinv_col(dgi_ref[...])
    do = _dinv_col(dgo_ref[...])
    x = x_ref[...]
    out_ref[0, :, :] = x * di
    out_ref[1, :, :] = x * do


def _prep_scaled(x, deg_in, deg_out):
    return pl.pallas_call(
        _prep_body,
        grid=(NUM_ENT // _RB,),
        in_specs=[
            pl.BlockSpec((_RB, DIM), lambda i: (i, 0)),
            pl.BlockSpec((_RB, DIM), lambda i: (i, 0)),
            pl.BlockSpec((_RB, DIM), lambda i: (i, 0)),
        ],
        out_specs=pl.BlockSpec((2, _RB, DIM), lambda i: (0, i, 0)),
        out_shape=jax.ShapeDtypeStruct((2, NUM_ENT, DIM), jnp.float32),
    )(x, deg_in, deg_out)


_BN_RS = 1.0 / math.sqrt(1.0 + 1e-5)


def _layer_body(emit_scaled, s_in_ref, s_out_ref, x_ref, dgi_ref, dgo_ref,
                w_in_ref, w_out_ref, w_loop_ref, lrel_ref, bias_ref,
                bng_ref, bnb_ref, *out_refs):
    di = _dinv_col(dgi_ref[...])
    do = _dinv_col(dgo_ref[...])
    x = x_ref[...]
    a = jnp.dot(s_in_ref[...] * di, w_in_ref[...],
                preferred_element_type=jnp.float32)
    a = a + jnp.dot(s_out_ref[...] * do, w_out_ref[...],
                    preferred_element_type=jnp.float32)
    a = a + jnp.dot(x * lrel_ref[...], w_loop_ref[...],
                    preferred_element_type=jnp.float32)
    h = a * (1.0 / 3.0) + bias_ref[...]
    h = (h * _BN_RS) * bng_ref[...] + bnb_ref[...]
    x2 = jnp.tanh(h)
    out_refs[0][...] = x2
    if emit_scaled:
        out_refs[1][0, :, :] = x2 * di
        out_refs[1][1, :, :] = x2 * do


def _layer(emit_scaled, s_in, s_out, x, deg_in, deg_out,
           w_in, w_out, w_loop, loop_rel, bias, bn_g, bn_b):
    full = lambda r, c: pl.BlockSpec((r, c), lambda i: (0, 0))
    out_shape = [jax.ShapeDtypeStruct((NUM_ENT, DIM), jnp.float32)]
    out_specs = [pl.BlockSpec((_RB, DIM), lambda i: (i, 0))]
    if emit_scaled:
        out_shape.append(jax.ShapeDtypeStruct((2, NUM_ENT, DIM), jnp.float32))
        out_specs.append(pl.BlockSpec((2, _RB, DIM), lambda i: (0, i, 0)))
    return pl.pallas_call(
        functools.partial(_layer_body, emit_scaled),
        grid=(NUM_ENT // _RB,),
        in_specs=[
            pl.BlockSpec((_RB, DIM), lambda i: (i, 0)),
            pl.BlockSpec((_RB, DIM), lambda i: (i, 0)),
            pl.BlockSpec((_RB, DIM), lambda i: (i, 0)),
            pl.BlockSpec((_RB, DIM), lambda i: (i, 0)),
            pl.BlockSpec((_RB, DIM), lambda i: (i, 0)),
            full(DIM, DIM), full(DIM, DIM), full(DIM, DIM),
            full(1, DIM), full(1, DIM), full(1, DIM), full(1, DIM),
        ],
        out_specs=out_specs,
        out_shape=out_shape,
    )(s_in, s_out, x, deg_in, deg_out, w_in, w_out, w_loop,
      loop_rel, bias, bn_g, bn_b)


def _relmm_body(r_ref, w_ref, o_ref):
    o_ref[...] = jnp.dot(r_ref[...], w_ref[...],
                         preferred_element_type=jnp.float32)


def _relmm(r, w):
    return pl.pallas_call(
        _relmm_body,
        out_shape=jax.ShapeDtypeStruct((NUM_RELROWS, DIM), jnp.float32),
    )(r, w)


def _prep_half(agg, src, typ, src_offset):
    def pad(a, fill):
        a = a.reshape(NS, EPT)
        a = jnp.pad(a, ((0, 0), (0, EPT_PAD - EPT)), constant_values=fill)
        return a.reshape(NS, CH, K)
    return (pad(agg, NUM_ENT), pad(src + src_offset, src_offset), pad(typ, 0))


def kernel(edge_index, edge_type, init_embed, init_rel,
           w_loop1, w_in1, w_out1, w_rel1,
           w_loop2, w_in2, w_out2, w_rel2,
           loop_rel1, loop_rel2, bias1, bias2,
           bn1_gamma, bn1_beta, bn2_gamma, bn2_beta):
    ei = edge_index.astype(jnp.int32)
    et = edge_type.astype(jnp.int32)
    agg_i, src_i, typ_i = _prep_half(ei[0, :NE_HALF], ei[1, :NE_HALF],
                                     et[:NE_HALF], 0)
    agg_o, src_o, typ_o = _prep_half(ei[0, NE_HALF:], ei[1, NE_HALF:],
                                     et[NE_HALF:], NUM_ENT)
    agg_all = jnp.stack([agg_i, agg_o])
    src_all = jnp.stack([src_i, src_o])
    typ_all = jnp.stack([typ_i, typ_o])

    deg2 = _deg_kernel(agg_all)
    deg_in = deg2[0, :NUM_ENT]
    deg_out = deg2[1, :NUM_ENT]

    b1 = bias1.reshape(1, DIM)
    b2 = bias2.reshape(1, DIM)
    g1 = bn1_gamma.reshape(1, DIM)
    be1 = bn1_beta.reshape(1, DIM)
    g2 = bn2_gamma.reshape(1, DIM)
    be2 = bn2_beta.reshape(1, DIM)

    xs1 = _prep_scaled(init_embed, deg_in, deg_out).reshape(2 * NUM_ENT, DIM)
    s1 = _agg_kernel(agg_all, src_all, typ_all, xs1, init_rel)
    x2, xs2 = _layer(True, s1[0, :NUM_ENT], s1[1, :NUM_ENT], init_embed,
                     deg_in, deg_out, w_in1, w_out1, w_loop1,
                     loop_rel1, b1, g1, be1)
    r1 = _relmm(init_rel, w_rel1)

    s2 = _agg_kernel(agg_all, src_all, typ_all,
                     xs2.reshape(2 * NUM_ENT, DIM), r1)
    (x3,) = _layer(False, s2[0, :NUM_ENT], s2[1, :NUM_ENT], x2,
                   deg_in, deg_out, w_in2, w_out2, w_loop2,
                   loop_rel2, b2, g2, be2)
    r2 = _relmm(r1, w_rel2)
    return x3, r2
